# speedup vs baseline: 2.2499x; 2.2499x over previous
import functools

import jax
import jax.numpy as jnp
from jax import lax
from jax.experimental import pallas as pl


def _adj_body(emb_ref, adj_ref):
    emb = emb_ref[...]
    sq = jnp.sum(emb * emb, axis=1, keepdims=True)
    nrm = emb * lax.rsqrt(jnp.maximum(sq, 1e-12))
    adj_ref[...] = lax.dot_general(
        nrm, nrm, (((1,), (1,)), ((), ())), preferred_element_type=jnp.float32
    )


def _gat_heads(adj_mask, h, e_dst, e_srcT, num_heads, head_dim):
    outs = []
    for hh in range(num_heads):
        d = e_dst[:, hh : hh + 1]
        s = e_srcT[hh : hh + 1, :]
        ev = d + s
        ev = jnp.where(ev > 0, ev, 0.2 * ev)
        xm = jnp.where(adj_mask, ev, -1e9)
        m = jnp.max(xm, axis=1, keepdims=True)
        p = jnp.exp(xm - m)
        denom = jnp.sum(p, axis=1, keepdims=True)
        oh = lax.dot_general(
            p,
            h[:, hh * head_dim : (hh + 1) * head_dim],
            (((1,), (0,)), ((), ())),
            preferred_element_type=jnp.float32,
        )
        outs.append(oh / denom)
    return jnp.concatenate(outs, axis=1)


def _gat_body(
    x_ref,
    adj_ref,
    wp_ref,
    bp_ref,
    w1_ref,
    a1s_ref,
    a1d_ref,
    b1_ref,
    w2_ref,
    a2s_ref,
    a2d_ref,
    b2_ref,
    out_ref,
    *,
    num_heads,
):
    adj_mask = adj_ref[...] > 0.5
    xb = x_ref[0]
    xp = (
        lax.dot_general(
            xb, wp_ref[...], (((1,), (0,)), ((), ())),
            preferred_element_type=jnp.float32,
        )
        + bp_ref[...]
    )

    def layer(inp, w_ref, as_ref, ad_ref):
        h = lax.dot_general(
            inp, w_ref[...], (((1,), (0,)), ((), ())),
            preferred_element_type=jnp.float32,
        )
        e_dst = lax.dot_general(
            h, ad_ref[...], (((1,), (0,)), ((), ())),
            preferred_element_type=jnp.float32,
        )
        e_srcT = lax.dot_general(
            as_ref[...], h, (((0,), (1,)), ((), ())),
            preferred_element_type=jnp.float32,
        )
        head_dim = h.shape[1] // num_heads
        return _gat_heads(adj_mask, h, e_dst, e_srcT, num_heads, head_dim)

    o1 = layer(xp, w1_ref, a1s_ref, a1d_ref)
    o1 = jnp.maximum(o1 + b1_ref[...], 0.0)
    o2 = layer(o1, w2_ref, a2s_ref, a2d_ref)
    out_ref[0] = o2 + b2_ref[...]


def _blockdiag(a):
    heads, dim = a.shape
    eye = jnp.eye(heads, dtype=a.dtype)
    return (a[:, :, None] * eye[:, None, :]).reshape(heads * dim, heads)


def kernel(x, embedding, W_proj, b_proj, W1, a1_src, a1_dst, b1, W2, a2_src, a2_dst, b2):
    batch, n, f_in = x.shape
    n_nodes, hidden = embedding.shape[0], W_proj.shape[1]
    heads = a1_src.shape[0]
    out_dim = W2.shape[1] * W2.shape[2]

    adj = pl.pallas_call(
        _adj_body,
        out_shape=jax.ShapeDtypeStruct((n_nodes, n_nodes), jnp.float32),
    )(embedding)

    w1r = W1.reshape(hidden, hidden)
    w2r = W2.reshape(hidden, out_dim)
    a1s = _blockdiag(a1_src)
    a1d = _blockdiag(a1_dst)
    a2s = _blockdiag(a2_src)
    a2d = _blockdiag(a2_dst)
    b_proj2 = b_proj.reshape(1, hidden)
    b1r = b1.reshape(1, hidden)
    b2r = b2.reshape(1, out_dim)

    full = lambda shape: pl.BlockSpec(shape, lambda b: (0,) * len(shape))

    out = pl.pallas_call(
        functools.partial(_gat_body, num_heads=heads),
        grid=(batch,),
        in_specs=[
            pl.BlockSpec((1, n, f_in), lambda b: (b, 0, 0)),
            full((n_nodes, n_nodes)),
            full((f_in, hidden)),
            full((1, hidden)),
            full((hidden, hidden)),
            full((hidden, heads)),
            full((hidden, heads)),
            full((1, hidden)),
            full((hidden, out_dim)),
            full((out_dim, heads)),
            full((out_dim, heads)),
            full((1, out_dim)),
        ],
        out_specs=pl.BlockSpec((1, n, out_dim), lambda b: (b, 0, 0)),
        out_shape=jax.ShapeDtypeStruct((batch, n, out_dim), jnp.float32),
    )(x, adj, W_proj, b_proj2, w1r, a1s, a1d, b1r, w2r, a2s, a2d, b2r)
    return out
